# baseline (speedup 1.0000x reference)
import functools
import math

import jax
import jax.numpy as jnp
from jax import lax
from jax.experimental import pallas as pl
from jax.experimental.pallas import tpu as pltpu
from jax.experimental.pallas import tpu_sc as plsc

D_MODEL = 64
N_POS = 16384
N_J = 50
SCALE = math.sqrt(D_MODEL)

_info = plsc.get_sparse_core_info()
_NC, _NS = _info.num_cores, _info.num_subcores
_NW = _NC * _NS
POS_PER_W = N_POS // _NW
CI = 64
N_BLK = POS_PER_W // CI


def _emb_body(xt_hbm, lut_hbm, outp_hbm, idx_v, rbufs, obufs, sems_g, sems_o):
    wid = lax.axis_index("s") * _NC + lax.axis_index("c")

    def gather_start(j, b):
        pltpu.async_copy(lut_hbm.at[idx_v.at[j]], rbufs[b], sems_g[b])

    def gather_wait(j, b):
        pltpu.make_async_copy(lut_hbm.at[idx_v.at[j]], rbufs[b], sems_g[b]).wait()

    def out_start(i0, j, b):
        pltpu.async_copy(obufs[b], outp_hbm.at[j, :, pl.ds(i0, CI)], sems_o[b])

    def out_wait(i0, j, b):
        pltpu.make_async_copy(
            obufs[b], outp_hbm.at[j, :, pl.ds(i0, CI)], sems_o[b]
        ).wait()

    lane = lax.iota(jnp.int32, 16)
    rowvs = [lane + c * 16 for c in range(CI // 16)]

    def transpose(b):
        def fbody(f, c2):
            colv = jnp.full((16,), f, jnp.int32)
            for c in range(CI // 16):
                vec = plsc.load_gather(rbufs[b], [rowvs[c], colv])
                obufs[b][f, pl.ds(c * 16, 16)] = vec * SCALE
            return c2

        lax.fori_loop(0, D_MODEL, fbody, 0)

    def block(t, carry):
        i0 = wid * POS_PER_W + t * CI
        pltpu.sync_copy(xt_hbm.at[:, pl.ds(i0, CI)], idx_v)

        def step(j, b):
            gather_wait(j, b)
            transpose(b)
            out_start(i0, j, b)

        gather_start(0, 0)
        gather_start(1, 1)
        for j in range(2):
            b = j % 2
            gather_wait(j, b)
            transpose(b)
            out_start(i0, j, b)
            gather_start(j + 2, b)

        def inner(jj, c2):
            for b in range(2):
                j = 2 * jj + b
                gather_wait(j, b)
                out_wait(i0, j - 2, b)
                transpose(b)
                out_start(i0, j, b)
                gather_start(j + 2, b)
            return c2

        lax.fori_loop(1, N_J // 2 - 1, inner, 0)

        for j in range(N_J - 2, N_J):
            b = j % 2
            gather_wait(j, b)
            out_wait(i0, j - 2, b)
            transpose(b)
            out_start(i0, j, b)

        out_wait(i0, N_J - 2, 0)
        out_wait(i0, N_J - 1, 1)
        return carry

    lax.fori_loop(0, N_BLK, block, 0)


_emb = functools.partial(
    pl.kernel,
    out_type=jax.ShapeDtypeStruct((N_J, D_MODEL, N_POS), jnp.float32),
    mesh=plsc.VectorSubcoreMesh(core_axis_name="c", subcore_axis_name="s"),
    scratch_types=[
        pltpu.VMEM((N_J, CI), jnp.int32),
        [pltpu.VMEM((CI, D_MODEL), jnp.float32) for _ in range(2)],
        [pltpu.VMEM((D_MODEL, CI), jnp.float32) for _ in range(2)],
        [pltpu.SemaphoreType.DMA for _ in range(2)],
        [pltpu.SemaphoreType.DMA for _ in range(2)],
    ],
    compiler_params=pltpu.CompilerParams(
        use_tc_tiling_on_sc=False, needs_layout_passes=False
    ),
)(_emb_body)


@jax.jit
def kernel(x, lut):
    xt = x.T.astype(jnp.int32)
    outp = _emb(xt, lut)
    return outp.transpose(2, 0, 1)


# speedup vs baseline: 1.0002x; 1.0002x over previous
import functools
import math

import jax
import jax.numpy as jnp
from jax import lax
from jax.experimental import pallas as pl
from jax.experimental.pallas import tpu as pltpu
from jax.experimental.pallas import tpu_sc as plsc

D_MODEL = 64
N_POS = 16384
N_J = 50
SCALE = math.sqrt(D_MODEL)

_info = plsc.get_sparse_core_info()
_NC, _NS = _info.num_cores, _info.num_subcores
_NW = _NC * _NS
POS_PER_W = N_POS // _NW
CI = 128
N_BLK = POS_PER_W // CI
NCH = CI // 16
F_UNROLL = 2


def _emb_body(xt_hbm, lut_hbm, outp_hbm, idx_v, rbufs, obufs, sems_g, sems_o):
    wid = lax.axis_index("s") * _NC + lax.axis_index("c")

    def gather_start(j, b):
        pltpu.async_copy(lut_hbm.at[idx_v.at[j]], rbufs[b], sems_g[b])

    def gather_wait(j, b):
        pltpu.make_async_copy(lut_hbm.at[idx_v.at[j]], rbufs[b], sems_g[b]).wait()

    def out_start(i0, j, b):
        pltpu.async_copy(obufs[b], outp_hbm.at[j, :, pl.ds(i0, CI)], sems_o[b])

    def out_wait(i0, j, b):
        pltpu.make_async_copy(
            obufs[b], outp_hbm.at[j, :, pl.ds(i0, CI)], sems_o[b]
        ).wait()

    lane = lax.iota(jnp.int32, 16)
    rowvs = [lane + c * 16 for c in range(NCH)]

    def transpose(b):
        def fbody(ff, c2):
            for u in range(F_UNROLL):
                f = ff * F_UNROLL + u
                colv = jnp.full((16,), f, jnp.int32)
                for c in range(NCH):
                    vec = plsc.load_gather(rbufs[b], [rowvs[c], colv])
                    obufs[b][f, pl.ds(c * 16, 16)] = vec * SCALE
            return c2

        lax.fori_loop(0, D_MODEL // F_UNROLL, fbody, 0)

    def block(t, carry):
        i0 = wid * POS_PER_W + t * CI
        pltpu.sync_copy(xt_hbm.at[:, pl.ds(i0, CI)], idx_v)

        gather_start(0, 0)
        gather_start(1, 1)
        for j in range(2):
            b = j % 2
            gather_wait(j, b)
            transpose(b)
            out_start(i0, j, b)
            gather_start(j + 2, b)

        def inner(jj, c2):
            for b in range(2):
                j = 2 * jj + b
                gather_wait(j, b)
                out_wait(i0, j - 2, b)
                transpose(b)
                out_start(i0, j, b)
                gather_start(j + 2, b)
            return c2

        lax.fori_loop(1, N_J // 2 - 1, inner, 0)

        for j in range(N_J - 2, N_J):
            b = j % 2
            gather_wait(j, b)
            out_wait(i0, j - 2, b)
            transpose(b)
            out_start(i0, j, b)

        out_wait(i0, N_J - 2, 0)
        out_wait(i0, N_J - 1, 1)
        return carry

    lax.fori_loop(0, N_BLK, block, 0)


_emb = functools.partial(
    pl.kernel,
    out_type=jax.ShapeDtypeStruct((N_J, D_MODEL, N_POS), jnp.float32),
    mesh=plsc.VectorSubcoreMesh(core_axis_name="c", subcore_axis_name="s"),
    scratch_types=[
        pltpu.VMEM((N_J, CI), jnp.int32),
        [pltpu.VMEM((CI, D_MODEL), jnp.float32) for _ in range(2)],
        [pltpu.VMEM((D_MODEL, CI), jnp.float32) for _ in range(2)],
        [pltpu.SemaphoreType.DMA for _ in range(2)],
        [pltpu.SemaphoreType.DMA for _ in range(2)],
    ],
    compiler_params=pltpu.CompilerParams(
        use_tc_tiling_on_sc=False, needs_layout_passes=False
    ),
)(_emb_body)


@jax.jit
def kernel(x, lut):
    xt = x.T.astype(jnp.int32)
    outp = _emb(xt, lut)
    return outp.transpose(2, 0, 1)


# speedup vs baseline: 1.4671x; 1.4668x over previous
import functools
import math

import jax
import jax.numpy as jnp
from jax import lax
from jax.experimental import pallas as pl
from jax.experimental.pallas import tpu as pltpu
from jax.experimental.pallas import tpu_sc as plsc

D_MODEL = 64
N_POS = 16384
N_J = 50
SCALE = math.sqrt(D_MODEL)

_info = plsc.get_sparse_core_info()
_NC, _NS = _info.num_cores, _info.num_subcores
_NW = _NC * _NS
POS_PER_W = N_POS // _NW
CI = 128
N_BLK = POS_PER_W // CI
NCH = CI // 16
F_UNROLL = 2


def _emb_body(xt_hbm, lut_hbm, outp_hbm, idx_v, rbufs, obufs, sems_g, sems_o):
    wid = lax.axis_index("s") * _NC + lax.axis_index("c")

    def gather_start(j, b):
        pltpu.async_copy(lut_hbm.at[idx_v.at[j]], rbufs[b], sems_g[b])

    def gather_wait(j, b):
        pltpu.make_async_copy(lut_hbm.at[idx_v.at[j]], rbufs[b], sems_g[b]).wait()

    def out_start(i0, j, b):
        pltpu.async_copy(obufs[b], outp_hbm.at[j, :, pl.ds(i0, CI)], sems_o[b])

    def out_wait(i0, j, b):
        pltpu.make_async_copy(
            obufs[b], outp_hbm.at[j, :, pl.ds(i0, CI)], sems_o[b]
        ).wait()

    lane = lax.iota(jnp.int32, 16)
    rowvs = [lane + c * 16 for c in range(NCH)]

    def transpose(b):
        @plsc.parallel_loop(0, D_MODEL, 1, unroll=F_UNROLL)
        def fbody(f):
            colv = jnp.full((16,), f, jnp.int32)
            for c in range(NCH):
                vec = plsc.load_gather(rbufs[b], [rowvs[c], colv])
                obufs[b][f, pl.ds(c * 16, 16)] = vec * SCALE

    def block(t, carry):
        i0 = wid * POS_PER_W + t * CI
        pltpu.sync_copy(xt_hbm.at[:, pl.ds(i0, CI)], idx_v)

        gather_start(0, 0)
        gather_start(1, 1)
        for j in range(2):
            b = j % 2
            gather_wait(j, b)
            transpose(b)
            out_start(i0, j, b)
            gather_start(j + 2, b)

        def inner(jj, c2):
            for b in range(2):
                j = 2 * jj + b
                gather_wait(j, b)
                out_wait(i0, j - 2, b)
                transpose(b)
                out_start(i0, j, b)
                gather_start(j + 2, b)
            return c2

        lax.fori_loop(1, N_J // 2 - 1, inner, 0)

        for j in range(N_J - 2, N_J):
            b = j % 2
            gather_wait(j, b)
            out_wait(i0, j - 2, b)
            transpose(b)
            out_start(i0, j, b)

        out_wait(i0, N_J - 2, 0)
        out_wait(i0, N_J - 1, 1)
        return carry

    lax.fori_loop(0, N_BLK, block, 0)


_emb = functools.partial(
    pl.kernel,
    out_type=jax.ShapeDtypeStruct((N_J, D_MODEL, N_POS), jnp.float32),
    mesh=plsc.VectorSubcoreMesh(core_axis_name="c", subcore_axis_name="s"),
    scratch_types=[
        pltpu.VMEM((N_J, CI), jnp.int32),
        [pltpu.VMEM((CI, D_MODEL), jnp.float32) for _ in range(2)],
        [pltpu.VMEM((D_MODEL, CI), jnp.float32) for _ in range(2)],
        [pltpu.SemaphoreType.DMA for _ in range(2)],
        [pltpu.SemaphoreType.DMA for _ in range(2)],
    ],
    compiler_params=pltpu.CompilerParams(
        use_tc_tiling_on_sc=False, needs_layout_passes=False
    ),
)(_emb_body)


@jax.jit
def kernel(x, lut):
    xt = x.T.astype(jnp.int32)
    outp = _emb(xt, lut)
    return outp.transpose(2, 0, 1)


# speedup vs baseline: 1.4691x; 1.0013x over previous
import functools
import math

import jax
import jax.numpy as jnp
from jax import lax
from jax.experimental import pallas as pl
from jax.experimental.pallas import tpu as pltpu
from jax.experimental.pallas import tpu_sc as plsc

D_MODEL = 64
N_POS = 16384
N_J = 50
SCALE = math.sqrt(D_MODEL)

_info = plsc.get_sparse_core_info()
_NC, _NS = _info.num_cores, _info.num_subcores
_NW = _NC * _NS
POS_PER_W = N_POS // _NW
CI = 256
N_BLK = POS_PER_W // CI
NCH = CI // 16
F_UNROLL = 2


def _emb_body(xt_hbm, lut_hbm, outp_hbm, idx_v, rbufs, obufs, sems_g, sems_o):
    wid = lax.axis_index("s") * _NC + lax.axis_index("c")

    def gather_start(j, b):
        for k in range(CI // 128):
            pltpu.async_copy(
                lut_hbm.at[idx_v.at[j, pl.ds(k * 128, 128)]],
                rbufs[b].at[pl.ds(k * 128, 128)],
                sems_g[b],
            )

    def gather_wait(j, b):
        for k in range(CI // 128):
            pltpu.make_async_copy(
                lut_hbm.at[idx_v.at[j, pl.ds(k * 128, 128)]],
                rbufs[b].at[pl.ds(k * 128, 128)],
                sems_g[b],
            ).wait()

    def out_start(i0, j, b):
        pltpu.async_copy(obufs[b], outp_hbm.at[j, :, pl.ds(i0, CI)], sems_o[b])

    def out_wait(i0, j, b):
        pltpu.make_async_copy(
            obufs[b], outp_hbm.at[j, :, pl.ds(i0, CI)], sems_o[b]
        ).wait()

    lane = lax.iota(jnp.int32, 16)
    rowvs = [lane + c * 16 for c in range(NCH)]

    def transpose(b):
        @plsc.parallel_loop(0, D_MODEL, 1, unroll=F_UNROLL)
        def fbody(f):
            colv = jnp.full((16,), f, jnp.int32)
            for c in range(NCH):
                vec = plsc.load_gather(rbufs[b], [rowvs[c], colv])
                obufs[b][f, pl.ds(c * 16, 16)] = vec * SCALE

    def block(t, carry):
        i0 = wid * POS_PER_W + t * CI
        pltpu.sync_copy(xt_hbm.at[:, pl.ds(i0, CI)], idx_v)

        gather_start(0, 0)
        gather_start(1, 1)
        for j in range(2):
            b = j % 2
            gather_wait(j, b)
            transpose(b)
            out_start(i0, j, b)
            gather_start(j + 2, b)

        def inner(jj, c2):
            for b in range(2):
                j = 2 * jj + b
                gather_wait(j, b)
                out_wait(i0, j - 2, b)
                transpose(b)
                out_start(i0, j, b)
                gather_start(j + 2, b)
            return c2

        lax.fori_loop(1, N_J // 2 - 1, inner, 0)

        for j in range(N_J - 2, N_J):
            b = j % 2
            gather_wait(j, b)
            out_wait(i0, j - 2, b)
            transpose(b)
            out_start(i0, j, b)

        out_wait(i0, N_J - 2, 0)
        out_wait(i0, N_J - 1, 1)
        return carry

    lax.fori_loop(0, N_BLK, block, 0)


_emb = functools.partial(
    pl.kernel,
    out_type=jax.ShapeDtypeStruct((N_J, D_MODEL, N_POS), jnp.float32),
    mesh=plsc.VectorSubcoreMesh(core_axis_name="c", subcore_axis_name="s"),
    scratch_types=[
        pltpu.VMEM((N_J, CI), jnp.int32),
        [pltpu.VMEM((CI, D_MODEL), jnp.float32) for _ in range(2)],
        [pltpu.VMEM((D_MODEL, CI), jnp.float32) for _ in range(2)],
        [pltpu.SemaphoreType.DMA for _ in range(2)],
        [pltpu.SemaphoreType.DMA for _ in range(2)],
    ],
    compiler_params=pltpu.CompilerParams(
        use_tc_tiling_on_sc=False, needs_layout_passes=False
    ),
)(_emb_body)


@jax.jit
def kernel(x, lut):
    xt = x.T.astype(jnp.int32)
    outp = _emb(xt, lut)
    return outp.transpose(2, 0, 1)
